# baseline (speedup 1.0000x reference)
import functools

import jax
import jax.numpy as jnp
from jax import lax
from jax.experimental import pallas as pl
from jax.experimental.pallas import tpu as pltpu
from jax.experimental.pallas import tpu_sc as plsc

B = 4
N = 500000
D = 16
GX = GY = GZ = 64
GC = GX * GY * GZ
CHUNK = 65536
NCHUNK = GC // CHUNK
NROUND = B * (NCHUNK // 2)
P = 1024
NBLK = (N + P - 1) // P
STEPS = (NBLK + 15) // 16
G = 1024
SCAP = G + P + 16
DUMP = 4096
GROWS = CHUNK + DUMP
TSLICE = GROWS // 16


def _sc_body(locs_hbm, data_hbm, dens_hbm, out_hbm,
             locs_v0, locs_v1, dens_v0, dens_v1,
             st_cell, st_rid, st_wgt, cellbuf, ridbuf, wbuf, gbuf, zbuf,
             grid_sh, wp_ref, pend_ref, b_ref,
             insem0, insem1, gsem):
    c = lax.axis_index("c")
    s = lax.axis_index("s")
    lane = lax.iota(jnp.int32, 16)
    locs_v = (locs_v0, locs_v1)
    dens_v = (dens_v0, dens_v1)
    insem = (insem0, insem1)

    def _z(i, _):
        zbuf[i, :] = jnp.zeros((16,), jnp.float32)
        return 0
    lax.fori_loop(0, P, _z, 0, unroll=4)
    wp_ref[0] = 0
    pend_ref[0] = 0
    b_ref[0] = 0

    def block_start(it):
        return jnp.minimum((it * 16 + s) * P, N - P)

    def drain_pending():
        @pl.when(pend_ref[0] == 1)
        def _():
            pltpu.make_async_copy(data_hbm.at[b_ref[0]].at[ridbuf], gbuf, gsem).wait()

            def _scale(g, _):
                wvec = wbuf[pl.ds(g * 16, 16)]
                for j in range(16):
                    i = g * 16 + j
                    gbuf[i, :] = gbuf[i, :] * wvec[j]
                return 0
            lax.fori_loop(0, G // 16, _scale, 0)
            pltpu.sync_copy(gbuf, grid_sh.at[cellbuf], add=True)
            pend_ref[0] = 0

    def compact_block(it, sl, my_chunk):
        s0 = (it * 16 + s) * P
        cs = block_start(it)
        lv, dv = locs_v[sl], dens_v[sl]

        def _group(g, wp):
            i16 = g * 16
            ivec = i16 + lane
            x = plsc.load_gather(lv, [ivec, jnp.zeros((16,), jnp.int32)])
            y = plsc.load_gather(lv, [ivec, jnp.full((16,), 1, jnp.int32)])
            z = plsc.load_gather(lv, [ivec, jnp.full((16,), 2, jnp.int32)])
            w = plsc.load_gather(lv, [ivec, jnp.full((16,), 3, jnp.int32)])
            dens = dv[pl.ds(i16, 16)]
            cx = jnp.clip((x * 64.0).astype(jnp.int32), 0, 63)
            cy = jnp.clip((y * 64.0).astype(jnp.int32), 0, 63)
            cz = jnp.clip((z * 64.0).astype(jnp.int32), 0, 63)
            flat = cx * 4096 + cy * 64 + cz
            valid = (cs + ivec) >= s0
            match = valid & (lax.shift_right_logical(flat, 16) == my_chunk)
            plsc.store_compressed(st_cell.at[pl.ds(wp, 16)],
                                  jnp.bitwise_and(flat, CHUNK - 1), mask=match)
            plsc.store_compressed(st_rid.at[pl.ds(wp, 16)],
                                  cs + ivec, mask=match)
            plsc.store_compressed(st_wgt.at[pl.ds(wp, 16)],
                                  1.0 / (w * dens), mask=match)
            cnt = plsc.all_reduce_population_count(match)
            return wp + cnt[0]
        wp = lax.fori_loop(0, P // 16, _group, wp_ref[0], unroll=2)
        wp_ref[0] = wp

    def maybe_fire():
        drain_pending()

        @pl.when(wp_ref[0] >= G)
        def _():
            def _cp(k, _):
                sel = pl.ds(k * 16, 16)
                cellbuf[sel] = st_cell[sel]
                ridbuf[sel] = st_rid[sel]
                wbuf[sel] = st_wgt[sel]
                return 0
            lax.fori_loop(0, G // 16, _cp, 0)
            wp = wp_ref[0]

            def _sh(k, _):
                dst = pl.ds(k * 16, 16)
                srcs = pl.ds(G + k * 16, 16)
                st_cell[dst] = st_cell[srcs]
                st_rid[dst] = st_rid[srcs]
                st_wgt[dst] = st_wgt[srcs]
                return 0
            lax.fori_loop(0, (wp - G + 15) // 16, _sh, 0)
            wp_ref[0] = wp - G
            pltpu.async_copy(data_hbm.at[b_ref[0]].at[ridbuf], gbuf, gsem)
            pend_ref[0] = 1

    def final_drain(my_chunk):
        drain_pending()

        @pl.when(wp_ref[0] > 0)
        def _():
            wp = wp_ref[0]

            def _pad(k, _):
                sel = pl.ds(k * 16, 16)
                inb = (k * 16 + lane) < wp
                dump = CHUNK + jnp.bitwise_and(k * 16 + lane, DUMP - 1)
                cellbuf[sel] = jnp.where(inb, st_cell[sel], dump)
                ridbuf[sel] = jnp.where(inb, st_rid[sel], 0)
                wbuf[sel] = st_wgt[sel]
                return 0
            lax.fori_loop(0, G // 16, _pad, 0)
            pltpu.async_copy(data_hbm.at[b_ref[0]].at[ridbuf], gbuf, gsem)
            pend_ref[0] = 1
            wp_ref[0] = 0
        drain_pending()

    def round_body(r, _):
        b = r // 2
        b_ref[0] = b
        my_chunk = (r % 2) * 2 + c

        zb = s * TSLICE
        for k in range(TSLICE // P):
            pltpu.sync_copy(zbuf, grid_sh.at[pl.ds(zb + k * P, P), :])
        pltpu.sync_copy(zbuf.at[pl.ds(0, TSLICE - (TSLICE // P) * P), :],
                        grid_sh.at[pl.ds(zb + (TSLICE // P) * P,
                                         TSLICE - (TSLICE // P) * P), :])
        plsc.subcore_barrier()

        def active(it):
            return (it * 16 + s) < NBLK

        def start_in(it, sl):
            @pl.when(active(it))
            def _():
                cs = block_start(it)
                pltpu.async_copy(locs_hbm.at[b, pl.ds(cs, P), :],
                                 locs_v[sl], insem[sl])
                pltpu.async_copy(dens_hbm.at[b, pl.ds(cs, P)],
                                 dens_v[sl], insem[sl])

        def wait_in(it, sl):
            @pl.when(active(it))
            def _():
                pltpu.make_async_copy(locs_hbm.at[0, pl.ds(0, P), :],
                                      locs_v[sl], insem[sl]).wait()
                pltpu.make_async_copy(dens_hbm.at[0, pl.ds(0, P)],
                                      dens_v[sl], insem[sl]).wait()

        start_in(0, 0)

        def pipe(i, _):
            for u in range(2):
                it = i * 2 + u
                start_in(it + 1, u ^ 1)
                wait_in(it, u)

                @pl.when(active(it))
                def _(it=it, u=u):
                    compact_block(it, u, my_chunk)
                maybe_fire()
            return 0
        lax.fori_loop(0, (STEPS + 1) // 2, pipe, 0)
        final_drain(my_chunk)
        plsc.subcore_barrier()

        base = b * GC + my_chunk * CHUNK + s * (CHUNK // 16)
        pltpu.sync_copy(grid_sh.at[pl.ds(s * (CHUNK // 16), CHUNK // 16), :],
                        out_hbm.at[pl.ds(base, CHUNK // 16), :])
        plsc.subcore_barrier()
        return 0

    lax.fori_loop(0, NROUND, round_body, 0)


@jax.jit
def _p2g(locs_f, data_f, dens_f):
    mesh = plsc.VectorSubcoreMesh(core_axis_name="c", subcore_axis_name="s")
    return pl.kernel(
        _sc_body,
        out_type=jax.ShapeDtypeStruct((B * GC, D), jnp.float32),
        mesh=mesh,
        compiler_params=pltpu.CompilerParams(
            needs_layout_passes=False, use_tc_tiling_on_sc=False),
        scratch_types=[
            pltpu.VMEM((P, 4), jnp.float32),
            pltpu.VMEM((P, 4), jnp.float32),
            pltpu.VMEM((P,), jnp.float32),
            pltpu.VMEM((P,), jnp.float32),
            pltpu.VMEM((SCAP,), jnp.int32),
            pltpu.VMEM((SCAP,), jnp.int32),
            pltpu.VMEM((SCAP,), jnp.float32),
            pltpu.VMEM((G,), jnp.int32),
            pltpu.VMEM((G,), jnp.int32),
            pltpu.VMEM((G,), jnp.float32),
            pltpu.VMEM((G, D), jnp.float32),
            pltpu.VMEM((P, D), jnp.float32),
            pltpu.VMEM_SHARED((GROWS, D), jnp.float32),
            pltpu.SMEM((1,), jnp.int32),
            pltpu.SMEM((1,), jnp.int32),
            pltpu.SMEM((1,), jnp.int32),
            pltpu.SemaphoreType.DMA,
            pltpu.SemaphoreType.DMA,
            pltpu.SemaphoreType.DMA,
        ],
    )(locs_f, data_f, dens_f)


def kernel(locs, data, density):
    out = _p2g(locs, data, density)
    return out.reshape(B, GX, GY, GZ, D)
